# Initial kernel scaffold; baseline (speedup 1.0000x reference)
import jax
import jax.numpy as jnp
from jax.experimental import pallas as pl


def kernel(x, edge_index, W1, b1, W2, b2, Wc, bc):
    raise NotImplementedError("write your pallas kernel here")


# speedup vs baseline: 61.9748x; 61.9748x over previous
import functools

import jax
import jax.numpy as jnp
from jax import lax
from jax.experimental import pallas as pl
from jax.experimental.pallas import tpu as pltpu
from jax.experimental.pallas import tpu_sc as plsc

N = 10000
E = 320000
D_IN = 128
H = 16
A = 10

NC = 2
NS = 16
NW = NC * NS
N_ACC = 10240
CHUNK = 2000
NWIN = E // (NW * CHUNK)
ROWS_PT = N_ACC // NS
BLK = 1000
GRID = N // BLK

_f32 = jnp.float32


@functools.cache
def _sc_kernels():
    mesh = plsc.VectorSubcoreMesh(core_axis_name="c", subcore_axis_name="s",
                                  num_cores=NC, num_subcores=NS)

    @functools.partial(
        pl.kernel,
        out_type=jax.ShapeDtypeStruct((NC, N_ACC, H), _f32),
        mesh=mesh,
        scratch_types=[
            pltpu.VMEM((NWIN, CHUNK), jnp.int32),
            pltpu.VMEM((CHUNK, H), _f32),
            pltpu.VMEM_SHARED((N_ACC, H), _f32),
            pltpu.SemaphoreType.DMA,
        ],
        compiler_params=pltpu.CompilerParams(use_tc_tiling_on_sc=False),
    )
    def sc_degree(edge_hbm, ones_hbm, zeros_hbm, out_hbm,
                  coli_v, ones_v, acc_sh, sem_s):
        cid = lax.axis_index("c")
        sid = lax.axis_index("s")
        wid = cid * NS + sid
        pltpu.sync_copy(edge_hbm.at[1, wid], coli_v)
        pltpu.sync_copy(ones_hbm, ones_v)
        pltpu.sync_copy(zeros_hbm, acc_sh.at[pl.ds(sid * ROWS_PT, ROWS_PT)])
        plsc.subcore_barrier()
        sds = [pltpu.async_copy(ones_v, acc_sh.at[coli_v.at[j]], sem_s,
                                add=True)
               for j in range(NWIN)]
        for sd in sds:
            sd.wait()
        plsc.subcore_barrier()
        pltpu.sync_copy(acc_sh.at[pl.ds(sid * ROWS_PT, ROWS_PT)],
                        out_hbm.at[cid, pl.ds(sid * ROWS_PT, ROWS_PT)])

    @functools.partial(
        pl.kernel,
        out_type=jax.ShapeDtypeStruct((NC, N_ACC, H), _f32),
        mesh=mesh,
        scratch_types=[
            pltpu.VMEM((NWIN, CHUNK), jnp.int32),
            pltpu.VMEM((NWIN, CHUNK), jnp.int32),
            pltpu.VMEM((CHUNK, H), _f32),
            pltpu.VMEM((CHUNK, H), _f32),
            pltpu.VMEM_SHARED((N_ACC, H), _f32),
            pltpu.SemaphoreType.DMA,
            pltpu.SemaphoreType.DMA,
            pltpu.SemaphoreType.DMA,
            pltpu.SemaphoreType.DMA,
        ],
        compiler_params=pltpu.CompilerParams(use_tc_tiling_on_sc=False),
    )
    def sc_scatter(y_hbm, edge_hbm, zeros_hbm, out_hbm,
                   rowi_v, coli_v, rows0_v, rows1_v, acc_sh,
                   sem_g0, sem_g1, sem_s0, sem_s1):
        cid = lax.axis_index("c")
        sid = lax.axis_index("s")
        wid = cid * NS + sid
        pltpu.sync_copy(edge_hbm.at[0, wid], rowi_v)
        pltpu.sync_copy(edge_hbm.at[1, wid], coli_v)
        pltpu.sync_copy(zeros_hbm, acc_sh.at[pl.ds(sid * ROWS_PT, ROWS_PT)])
        plsc.subcore_barrier()
        rows = (rows0_v, rows1_v)
        sem_g = (sem_g0, sem_g1)
        sem_s = (sem_s0, sem_s1)
        gd = [None] * NWIN
        sd = [None] * NWIN
        gd[0] = pltpu.async_copy(y_hbm.at[rowi_v.at[0]], rows[0], sem_g[0])
        for j in range(NWIN):
            gd[j].wait()
            sd[j] = pltpu.async_copy(rows[j % 2], acc_sh.at[coli_v.at[j]],
                                     sem_s[j % 2], add=True)
            if j + 1 < NWIN:
                if j >= 1:
                    sd[j - 1].wait()
                gd[j + 1] = pltpu.async_copy(y_hbm.at[rowi_v.at[j + 1]],
                                             rows[(j + 1) % 2],
                                             sem_g[(j + 1) % 2])
        sd[NWIN - 2].wait()
        sd[NWIN - 1].wait()
        plsc.subcore_barrier()
        pltpu.sync_copy(acc_sh.at[pl.ds(sid * ROWS_PT, ROWS_PT)],
                        out_hbm.at[cid, pl.ds(sid * ROWS_PT, ROWS_PT)])

    return sc_degree, sc_scatter


def _tc_scale_body(x_ref, w1_ref, degp_ref, y1_ref, dis_ref):
    deg = degp_ref[0] + degp_ref[1] + 1.0
    dis = lax.rsqrt(deg)
    xw = jnp.dot(x_ref[...], w1_ref[...], preferred_element_type=_f32)
    y1_ref[...] = xw * dis
    dis_ref[...] = dis


_tc_scale = pl.pallas_call(
    _tc_scale_body,
    grid=(GRID,),
    in_specs=[
        pl.BlockSpec((BLK, D_IN), lambda i: (i, 0)),
        pl.BlockSpec((D_IN, H), lambda i: (0, 0)),
        pl.BlockSpec((NC, BLK, H), lambda i: (0, i, 0)),
    ],
    out_specs=[
        pl.BlockSpec((BLK, H), lambda i: (i, 0)),
        pl.BlockSpec((BLK, H), lambda i: (i, 0)),
    ],
    out_shape=[
        jax.ShapeDtypeStruct((N, H), _f32),
        jax.ShapeDtypeStruct((N, H), _f32),
    ],
    compiler_params=pltpu.CompilerParams(dimension_semantics=("parallel",)),
)


def _tc_mid_body(sp_ref, y1_ref, dis_ref, b1_ref, w2_ref, y2_ref):
    s = sp_ref[0] + sp_ref[1] + y1_ref[...]
    h1 = jnp.maximum(dis_ref[...] * s + b1_ref[...], 0.0)
    y2_ref[...] = jnp.dot(h1, w2_ref[...], preferred_element_type=_f32) * dis_ref[...]


_tc_mid = pl.pallas_call(
    _tc_mid_body,
    grid=(GRID,),
    in_specs=[
        pl.BlockSpec((NC, BLK, H), lambda i: (0, i, 0)),
        pl.BlockSpec((BLK, H), lambda i: (i, 0)),
        pl.BlockSpec((BLK, H), lambda i: (i, 0)),
        pl.BlockSpec((1, H), lambda i: (0, 0)),
        pl.BlockSpec((H, H), lambda i: (0, 0)),
    ],
    out_specs=pl.BlockSpec((BLK, H), lambda i: (i, 0)),
    out_shape=jax.ShapeDtypeStruct((N, H), _f32),
    compiler_params=pltpu.CompilerParams(dimension_semantics=("parallel",)),
)


def _tc_out_body(sp_ref, y2_ref, dis_ref, b2_ref, wc_ref, bc_ref,
                 out_ref, acc_ref):
    i = pl.program_id(0)
    s = sp_ref[0] + sp_ref[1] + y2_ref[...]
    h2 = jnp.maximum(dis_ref[...] * s + b2_ref[...], 0.0)
    bmax = jnp.max(h2, axis=0, keepdims=True)

    @pl.when(i == 0)
    def _():
        acc_ref[...] = jnp.full((1, H), -jnp.inf, _f32)
        out_ref[...] = jnp.zeros((1, A), _f32)

    acc_ref[...] = jnp.maximum(acc_ref[...], bmax)

    @pl.when(i == pl.num_programs(0) - 1)
    def _():
        out_ref[...] = (jnp.dot(acc_ref[...], wc_ref[...],
                                preferred_element_type=_f32) + bc_ref[...])


_tc_out = pl.pallas_call(
    _tc_out_body,
    grid=(GRID,),
    in_specs=[
        pl.BlockSpec((NC, BLK, H), lambda i: (0, i, 0)),
        pl.BlockSpec((BLK, H), lambda i: (i, 0)),
        pl.BlockSpec((BLK, H), lambda i: (i, 0)),
        pl.BlockSpec((1, H), lambda i: (0, 0)),
        pl.BlockSpec((H, A), lambda i: (0, 0)),
        pl.BlockSpec((1, A), lambda i: (0, 0)),
    ],
    out_specs=pl.BlockSpec((1, A), lambda i: (0, 0)),
    out_shape=jax.ShapeDtypeStruct((1, A), _f32),
    scratch_shapes=[pltpu.VMEM((1, H), _f32)],
    compiler_params=pltpu.CompilerParams(dimension_semantics=("arbitrary",)),
)


def kernel(x, edge_index, W1, b1, W2, b2, Wc, bc):
    edge3 = edge_index.astype(jnp.int32).reshape(2, NW, NWIN, CHUNK)
    ones_c = jnp.ones((CHUNK, H), _f32)
    zeros_c = jnp.zeros((ROWS_PT, H), _f32)

    sc_degree, sc_scatter = _sc_kernels()
    degp = sc_degree(edge3, ones_c, zeros_c)
    y1, dis = _tc_scale(x, W1, degp)
    s1 = sc_scatter(y1, edge3, zeros_c)
    y2 = _tc_mid(s1, y1, dis, b1.reshape(1, H), W2)
    s2 = sc_scatter(y2, edge3, zeros_c)
    out = _tc_out(s2, y2, dis, b2.reshape(1, H), Wc, bc.reshape(1, A))
    return out.reshape((A,))


# speedup vs baseline: 89.7453x; 1.4481x over previous
import functools

import jax
import jax.numpy as jnp
from jax import lax
from jax.experimental import pallas as pl
from jax.experimental.pallas import tpu as pltpu
from jax.experimental.pallas import tpu_sc as plsc

N = 10000
E = 320000
D_IN = 128
H = 16
A = 10

NC = 2
NS = 16
NW = NC * NS
CHUNK = 2000
NWIN = E // (NW * CHUNK)
EPW = E // NW
ROWS_PT = N // NS
NF = (N * H) // 128

_f32 = jnp.float32


@functools.cache
def _sc_kernels():
    mesh = plsc.VectorSubcoreMesh(core_axis_name="c", subcore_axis_name="s",
                                  num_cores=NC, num_subcores=NS)

    @functools.partial(
        pl.kernel,
        out_type=jax.ShapeDtypeStruct((NC, N, H), _f32),
        mesh=mesh,
        scratch_types=[
            pltpu.VMEM((NWIN, CHUNK), jnp.int32),
            pltpu.VMEM((CHUNK, H), _f32),
            pltpu.VMEM_SHARED((N, H), _f32),
            pltpu.SemaphoreType.DMA,
            pltpu.SemaphoreType.DMA,
        ],
        compiler_params=pltpu.CompilerParams(use_tc_tiling_on_sc=False),
    )
    def sc_degree(edge_hbm, ones_hbm, zeros_hbm, out_hbm,
                  coli_v, ones_v, acc_sh, sem_i, sem_s):
        cid = lax.axis_index("c")
        sid = lax.axis_index("s")
        wid = cid * NS + sid
        base = wid * EPW
        idx_d = [pltpu.async_copy(edge_hbm.at[1, pl.ds(base + j * CHUNK, CHUNK)],
                                  coli_v.at[j], sem_i)
                 for j in range(NWIN)]
        pltpu.sync_copy(ones_hbm, ones_v)
        pltpu.sync_copy(zeros_hbm, acc_sh.at[pl.ds(sid * ROWS_PT, ROWS_PT)])
        for d in idx_d:
            d.wait()
        plsc.subcore_barrier()
        sds = [pltpu.async_copy(ones_v, acc_sh.at[coli_v.at[j]], sem_s,
                                add=True)
               for j in range(NWIN)]
        for sd in sds:
            sd.wait()
        plsc.subcore_barrier()
        pltpu.sync_copy(acc_sh.at[pl.ds(sid * ROWS_PT, ROWS_PT)],
                        out_hbm.at[cid, pl.ds(sid * ROWS_PT, ROWS_PT)])

    @functools.partial(
        pl.kernel,
        out_type=jax.ShapeDtypeStruct((NC, N, H), _f32),
        mesh=mesh,
        scratch_types=[
            pltpu.VMEM((NWIN, CHUNK), jnp.int32),
            pltpu.VMEM((NWIN, CHUNK), jnp.int32),
            pltpu.VMEM((CHUNK, H), _f32),
            pltpu.VMEM((CHUNK, H), _f32),
            pltpu.VMEM_SHARED((N, H), _f32),
            pltpu.SemaphoreType.DMA,
            pltpu.SemaphoreType.DMA,
            pltpu.SemaphoreType.DMA,
            pltpu.SemaphoreType.DMA,
            pltpu.SemaphoreType.DMA,
        ],
        compiler_params=pltpu.CompilerParams(use_tc_tiling_on_sc=False),
    )
    def sc_scatter(y_hbm, edge_hbm, zeros_hbm, out_hbm,
                   rowi_v, coli_v, rows0_v, rows1_v, acc_sh,
                   sem_i, sem_g0, sem_g1, sem_s0, sem_s1):
        cid = lax.axis_index("c")
        sid = lax.axis_index("s")
        wid = cid * NS + sid
        base = wid * EPW
        idx_d = [pltpu.async_copy(edge_hbm.at[r, pl.ds(base + j * CHUNK, CHUNK)],
                                  (rowi_v, coli_v)[r].at[j], sem_i)
                 for j in range(NWIN) for r in (0, 1)]
        pltpu.sync_copy(zeros_hbm, acc_sh.at[pl.ds(sid * ROWS_PT, ROWS_PT)])
        for d in idx_d:
            d.wait()
        plsc.subcore_barrier()
        rows = (rows0_v, rows1_v)
        sem_g = (sem_g0, sem_g1)
        sem_s = (sem_s0, sem_s1)
        gd = [None] * NWIN
        sd = [None] * NWIN
        gd[0] = pltpu.async_copy(y_hbm.at[rowi_v.at[0]], rows[0], sem_g[0])
        for j in range(NWIN):
            gd[j].wait()
            sd[j] = pltpu.async_copy(rows[j % 2], acc_sh.at[coli_v.at[j]],
                                     sem_s[j % 2], add=True)
            if j + 1 < NWIN:
                if j >= 1:
                    sd[j - 1].wait()
                gd[j + 1] = pltpu.async_copy(y_hbm.at[rowi_v.at[j + 1]],
                                             rows[(j + 1) % 2],
                                             sem_g[(j + 1) % 2])
        sd[NWIN - 2].wait()
        sd[NWIN - 1].wait()
        plsc.subcore_barrier()
        pltpu.sync_copy(acc_sh.at[pl.ds(sid * ROWS_PT, ROWS_PT)],
                        out_hbm.at[cid, pl.ds(sid * ROWS_PT, ROWS_PT)])

    return sc_degree, sc_scatter


def _tc_scale_body(xg_ref, w1big_ref, degp_ref, y1_ref, dis_ref):
    deg = degp_ref[0] + degp_ref[1] + 1.0
    dis = lax.rsqrt(deg)
    xw = jnp.dot(xg_ref[...], w1big_ref[...], preferred_element_type=_f32)
    y1_ref[...] = xw * dis
    dis_ref[...] = dis


_tc_scale = pl.pallas_call(
    _tc_scale_body,
    grid=(1,),
    in_specs=[
        pl.BlockSpec((NF, 8 * D_IN), lambda i: (0, 0)),
        pl.BlockSpec((8 * D_IN, 128), lambda i: (0, 0)),
        pl.BlockSpec((NC, NF, 128), lambda i: (0, 0, 0)),
    ],
    out_specs=[
        pl.BlockSpec((NF, 128), lambda i: (0, 0)),
        pl.BlockSpec((NF, 128), lambda i: (0, 0)),
    ],
    out_shape=[
        jax.ShapeDtypeStruct((NF, 128), _f32),
        jax.ShapeDtypeStruct((NF, 128), _f32),
    ],
)


def _tc_mid_body(sp_ref, y1_ref, dis_ref, b1_ref, w2big_ref, y2_ref):
    s = sp_ref[0] + sp_ref[1] + y1_ref[...]
    h1 = jnp.maximum(dis_ref[...] * s + b1_ref[...], 0.0)
    y2_ref[...] = jnp.dot(h1, w2big_ref[...],
                          preferred_element_type=_f32) * dis_ref[...]


_tc_mid = pl.pallas_call(
    _tc_mid_body,
    grid=(1,),
    in_specs=[
        pl.BlockSpec((NC, NF, 128), lambda i: (0, 0, 0)),
        pl.BlockSpec((NF, 128), lambda i: (0, 0)),
        pl.BlockSpec((NF, 128), lambda i: (0, 0)),
        pl.BlockSpec((1, 128), lambda i: (0, 0)),
        pl.BlockSpec((128, 128), lambda i: (0, 0)),
    ],
    out_specs=pl.BlockSpec((NF, 128), lambda i: (0, 0)),
    out_shape=jax.ShapeDtypeStruct((NF, 128), _f32),
)


def _tc_out_body(sp_ref, y2_ref, dis_ref, b2_ref, wc_ref, bc_ref, out_ref):
    s = sp_ref[0] + sp_ref[1] + y2_ref[...]
    h2 = jnp.maximum(dis_ref[...] * s + b2_ref[...], 0.0)
    m = jnp.max(h2, axis=0, keepdims=True)
    pooled = m[:, 0:H]
    for a in range(1, 8):
        pooled = jnp.maximum(pooled, m[:, a * H:(a + 1) * H])
    out_ref[...] = (jnp.dot(pooled, wc_ref[...],
                            preferred_element_type=_f32) + bc_ref[...])


_tc_out = pl.pallas_call(
    _tc_out_body,
    grid=(1,),
    in_specs=[
        pl.BlockSpec((NC, NF, 128), lambda i: (0, 0, 0)),
        pl.BlockSpec((NF, 128), lambda i: (0, 0)),
        pl.BlockSpec((NF, 128), lambda i: (0, 0)),
        pl.BlockSpec((1, 128), lambda i: (0, 0)),
        pl.BlockSpec((H, A), lambda i: (0, 0)),
        pl.BlockSpec((1, A), lambda i: (0, 0)),
    ],
    out_specs=pl.BlockSpec((1, A), lambda i: (0, 0)),
    out_shape=jax.ShapeDtypeStruct((1, A), _f32),
)


def kernel(x, edge_index, W1, b1, W2, b2, Wc, bc):
    edge = edge_index.astype(jnp.int32)
    xg = x.reshape(NF, 8 * D_IN)
    eye8 = jnp.eye(8, dtype=_f32)
    w1big = jnp.kron(eye8, W1)
    w2big = jnp.kron(eye8, W2)
    b1f = jnp.tile(b1, 8).reshape(1, 128)
    b2f = jnp.tile(b2, 8).reshape(1, 128)
    ones_c = jnp.ones((CHUNK, H), _f32)
    zeros_c = jnp.zeros((ROWS_PT, H), _f32)

    sc_degree, sc_scatter = _sc_kernels()
    degp = sc_degree(edge, ones_c, zeros_c)
    y1f, disf = _tc_scale(xg, w1big, degp.reshape(NC, NF, 128))
    s1 = sc_scatter(y1f.reshape(N, H), edge, zeros_c)
    y2f = _tc_mid(s1.reshape(NC, NF, 128), y1f, disf, b1f, w2big)
    s2 = sc_scatter(y2f.reshape(N, H), edge, zeros_c)
    out = _tc_out(s2.reshape(NC, NF, 128), y2f, disf, b2f,
                  Wc, bc.reshape(1, A))
    return out.reshape((A,))


# speedup vs baseline: 92.3019x; 1.0285x over previous
import functools

import jax
import jax.numpy as jnp
from jax import lax
from jax.experimental import pallas as pl
from jax.experimental.pallas import tpu as pltpu
from jax.experimental.pallas import tpu_sc as plsc

N = 10000
E = 320000
D_IN = 128
H = 16
A = 10

NC = 2
NS = 16
NW = NC * NS
CHUNK = 2000
NWIN = E // (NW * CHUNK)
EPW = E // NW
ROWS_PT = N // NS
NF = (N * H) // 128

_f32 = jnp.float32


@functools.cache
def _sc_kernels():
    mesh = plsc.VectorSubcoreMesh(core_axis_name="c", subcore_axis_name="s",
                                  num_cores=NC, num_subcores=NS)

    @functools.partial(
        pl.kernel,
        out_type=jax.ShapeDtypeStruct((NC, N, H), _f32),
        mesh=mesh,
        scratch_types=[
            pltpu.VMEM((NWIN, CHUNK), jnp.int32),
            pltpu.VMEM((CHUNK, H), _f32),
            pltpu.VMEM_SHARED((N, H), _f32),
            pltpu.SemaphoreType.DMA,
            pltpu.SemaphoreType.DMA,
        ],
        compiler_params=pltpu.CompilerParams(use_tc_tiling_on_sc=False),
    )
    def sc_degree(edge_hbm, ones_hbm, zeros_hbm, out_hbm,
                  coli_v, ones_v, acc_sh, sem_i, sem_s):
        cid = lax.axis_index("c")
        sid = lax.axis_index("s")
        wid = cid * NS + sid
        base = wid * EPW
        idx_d = [pltpu.async_copy(edge_hbm.at[1, pl.ds(base + j * CHUNK, CHUNK)],
                                  coli_v.at[j], sem_i)
                 for j in range(NWIN)]
        pltpu.sync_copy(ones_hbm, ones_v)
        pltpu.sync_copy(zeros_hbm, acc_sh.at[pl.ds(sid * ROWS_PT, ROWS_PT)])
        for d in idx_d:
            d.wait()
        plsc.subcore_barrier()
        sds = [pltpu.async_copy(ones_v, acc_sh.at[coli_v.at[j]], sem_s,
                                add=True)
               for j in range(NWIN)]
        for sd in sds:
            sd.wait()
        plsc.subcore_barrier()
        pltpu.sync_copy(acc_sh.at[pl.ds(sid * ROWS_PT, ROWS_PT)],
                        out_hbm.at[cid, pl.ds(sid * ROWS_PT, ROWS_PT)])

    @functools.partial(
        pl.kernel,
        out_type=jax.ShapeDtypeStruct((NC, N, H), _f32),
        mesh=mesh,
        scratch_types=[
            pltpu.VMEM((NWIN, CHUNK), jnp.int32),
            pltpu.VMEM((NWIN, CHUNK), jnp.int32),
            pltpu.VMEM((CHUNK, H), _f32),
            pltpu.VMEM((CHUNK, H), _f32),
            pltpu.VMEM((CHUNK, H), _f32),
            pltpu.VMEM_SHARED((N, H), _f32),
            pltpu.SemaphoreType.DMA,
            pltpu.SemaphoreType.DMA,
            pltpu.SemaphoreType.DMA,
            pltpu.SemaphoreType.DMA,
            pltpu.SemaphoreType.DMA,
            pltpu.SemaphoreType.DMA,
            pltpu.SemaphoreType.DMA,
        ],
        compiler_params=pltpu.CompilerParams(use_tc_tiling_on_sc=False),
    )
    def sc_scatter(y_hbm, edge_hbm, zeros_hbm, out_hbm,
                   rowi_v, coli_v, rows0_v, rows1_v, rows2_v, acc_sh,
                   sem_i, sem_g0, sem_g1, sem_g2, sem_s0, sem_s1, sem_s2):
        cid = lax.axis_index("c")
        sid = lax.axis_index("s")
        wid = cid * NS + sid
        base = wid * EPW
        idx_d = [pltpu.async_copy(edge_hbm.at[r, pl.ds(base + j * CHUNK, CHUNK)],
                                  (rowi_v, coli_v)[r].at[j], sem_i)
                 for j in range(NWIN) for r in (0, 1)]
        pltpu.sync_copy(zeros_hbm, acc_sh.at[pl.ds(sid * ROWS_PT, ROWS_PT)])
        for d in idx_d:
            d.wait()
        plsc.subcore_barrier()
        rows = (rows0_v, rows1_v, rows2_v)
        sem_g = (sem_g0, sem_g1, sem_g2)
        sem_s = (sem_s0, sem_s1, sem_s2)
        gd = [None] * NWIN
        sd = [None] * NWIN
        for j in range(2):
            gd[j] = pltpu.async_copy(y_hbm.at[rowi_v.at[j]], rows[j],
                                     sem_g[j])
        for j in range(NWIN):
            gd[j].wait()
            sd[j] = pltpu.async_copy(rows[j % 3], acc_sh.at[coli_v.at[j]],
                                     sem_s[j % 3], add=True)
            if j + 2 < NWIN:
                if j >= 1:
                    sd[j - 1].wait()
                gd[j + 2] = pltpu.async_copy(y_hbm.at[rowi_v.at[j + 2]],
                                             rows[(j + 2) % 3],
                                             sem_g[(j + 2) % 3])
        sd[NWIN - 3].wait()
        sd[NWIN - 2].wait()
        sd[NWIN - 1].wait()
        plsc.subcore_barrier()
        pltpu.sync_copy(acc_sh.at[pl.ds(sid * ROWS_PT, ROWS_PT)],
                        out_hbm.at[cid, pl.ds(sid * ROWS_PT, ROWS_PT)])

    return sc_degree, sc_scatter


def _tc_mm_body(xg_ref, w1big_ref, xw_ref):
    xw_ref[...] = jnp.dot(xg_ref[...], w1big_ref[...],
                          preferred_element_type=_f32)


_tc_mm = pl.pallas_call(
    _tc_mm_body,
    grid=(1,),
    in_specs=[
        pl.BlockSpec((NF, 8 * D_IN), lambda i: (0, 0)),
        pl.BlockSpec((8 * D_IN, 128), lambda i: (0, 0)),
    ],
    out_specs=pl.BlockSpec((NF, 128), lambda i: (0, 0)),
    out_shape=jax.ShapeDtypeStruct((NF, 128), _f32),
)


def _tc_scale_body(xw_ref, degp_ref, y1_ref, dis_ref):
    deg = degp_ref[0] + degp_ref[1] + 1.0
    dis = lax.rsqrt(deg)
    y1_ref[...] = xw_ref[...] * dis
    dis_ref[...] = dis


_tc_scale = pl.pallas_call(
    _tc_scale_body,
    grid=(1,),
    in_specs=[
        pl.BlockSpec((NF, 128), lambda i: (0, 0)),
        pl.BlockSpec((NC, NF, 128), lambda i: (0, 0, 0)),
    ],
    out_specs=[
        pl.BlockSpec((NF, 128), lambda i: (0, 0)),
        pl.BlockSpec((NF, 128), lambda i: (0, 0)),
    ],
    out_shape=[
        jax.ShapeDtypeStruct((NF, 128), _f32),
        jax.ShapeDtypeStruct((NF, 128), _f32),
    ],
)


def _tc_mid_body(sp_ref, y1_ref, dis_ref, b1_ref, w2big_ref, y2_ref):
    s = sp_ref[0] + sp_ref[1] + y1_ref[...]
    h1 = jnp.maximum(dis_ref[...] * s + b1_ref[...], 0.0)
    y2_ref[...] = jnp.dot(h1, w2big_ref[...],
                          preferred_element_type=_f32) * dis_ref[...]


_tc_mid = pl.pallas_call(
    _tc_mid_body,
    grid=(1,),
    in_specs=[
        pl.BlockSpec((NC, NF, 128), lambda i: (0, 0, 0)),
        pl.BlockSpec((NF, 128), lambda i: (0, 0)),
        pl.BlockSpec((NF, 128), lambda i: (0, 0)),
        pl.BlockSpec((1, 128), lambda i: (0, 0)),
        pl.BlockSpec((128, 128), lambda i: (0, 0)),
    ],
    out_specs=pl.BlockSpec((NF, 128), lambda i: (0, 0)),
    out_shape=jax.ShapeDtypeStruct((NF, 128), _f32),
)


def _tc_out_body(sp_ref, y2_ref, dis_ref, b2_ref, wc_ref, bc_ref, out_ref):
    s = sp_ref[0] + sp_ref[1] + y2_ref[...]
    h2 = jnp.maximum(dis_ref[...] * s + b2_ref[...], 0.0)
    m = jnp.max(h2, axis=0, keepdims=True)
    pooled = m[:, 0:H]
    for a in range(1, 8):
        pooled = jnp.maximum(pooled, m[:, a * H:(a + 1) * H])
    out_ref[...] = (jnp.dot(pooled, wc_ref[...],
                            preferred_element_type=_f32) + bc_ref[...])


_tc_out = pl.pallas_call(
    _tc_out_body,
    grid=(1,),
    in_specs=[
        pl.BlockSpec((NC, NF, 128), lambda i: (0, 0, 0)),
        pl.BlockSpec((NF, 128), lambda i: (0, 0)),
        pl.BlockSpec((NF, 128), lambda i: (0, 0)),
        pl.BlockSpec((1, 128), lambda i: (0, 0)),
        pl.BlockSpec((H, A), lambda i: (0, 0)),
        pl.BlockSpec((1, A), lambda i: (0, 0)),
    ],
    out_specs=pl.BlockSpec((1, A), lambda i: (0, 0)),
    out_shape=jax.ShapeDtypeStruct((1, A), _f32),
)


def kernel(x, edge_index, W1, b1, W2, b2, Wc, bc):
    edge = edge_index.astype(jnp.int32)
    xg = x.reshape(NF, 8 * D_IN)
    eye8 = jnp.eye(8, dtype=_f32)
    w1big = jnp.kron(eye8, W1)
    w2big = jnp.kron(eye8, W2)
    b1f = jnp.tile(b1, 8).reshape(1, 128)
    b2f = jnp.tile(b2, 8).reshape(1, 128)
    ones_c = jnp.ones((CHUNK, H), _f32)
    zeros_c = jnp.zeros((ROWS_PT, H), _f32)

    sc_degree, sc_scatter = _sc_kernels()
    degp = sc_degree(edge, ones_c, zeros_c)
    xwf = _tc_mm(xg, w1big)
    y1f, disf = _tc_scale(xwf, degp.reshape(NC, NF, 128))
    s1 = sc_scatter(y1f.reshape(N, H), edge, zeros_c)
    y2f = _tc_mid(s1.reshape(NC, NF, 128), y1f, disf, b1f, w2big)
    s2 = sc_scatter(y2f.reshape(N, H), edge, zeros_c)
    out = _tc_out(s2.reshape(NC, NF, 128), y2f, disf, b2f,
                  Wc, bc.reshape(1, A))
    return out.reshape((A,))
